# baseline (speedup 1.0000x reference)
import functools

import jax
import jax.numpy as jnp
from jax import lax
from jax.experimental import pallas as pl
from jax.experimental.pallas import tpu as pltpu
from jax.experimental.pallas import tpu_sc as plsc

N_NODES = 10000
N_EDGES = 320000
D_FEAT = 128

NC = 2
NS = 16
NW = NC * NS
CHUNK = 64
NBUF = 4
GROUPS = 40
E_PAD = NW * GROUPS * NBUF * CHUNK
SD_GROUPS = GROUPS + 2
ROWS_PER_TILE = 632
N_ACC = ROWS_PER_TILE * NS
JUNK = N_NODES
L = 16


def _sc_accumulate(x, sd3, zeros_init):
    mesh = plsc.VectorSubcoreMesh(core_axis_name="c", subcore_axis_name="s")

    @functools.partial(
        pl.kernel,
        out_type=(
            jax.ShapeDtypeStruct((NC, N_ACC, D_FEAT), jnp.float32),
            jax.ShapeDtypeStruct((NC, NS, N_ACC), jnp.float32),
        ),
        mesh=mesh,
        scratch_types=[
            pltpu.VMEM((3, 2 * NBUF, CHUNK), jnp.int32),
            [pltpu.VMEM((CHUNK, D_FEAT), jnp.float32) for _ in range(NBUF)],
            pltpu.VMEM((N_ACC,), jnp.float32),
            pltpu.VMEM_SHARED((N_ACC, D_FEAT), jnp.float32),
            [pltpu.SemaphoreType.DMA for _ in range(NBUF)],
            pltpu.SemaphoreType.DMA,
        ],
        compiler_params=pltpu.CompilerParams(needs_layout_passes=False),
    )
    def sc_kernel(x_hbm, sd_hbm, zero_hbm, acc_hbm, deg_hbm,
                  sd_w, bufs, hist_v, acc_sh, sems, sem_i):
        c = lax.axis_index("c")
        s = lax.axis_index("s")
        wid = s * NC + c

        pltpu.sync_copy(zero_hbm,
                        acc_sh.at[pl.ds(s * ROWS_PER_TILE, ROWS_PER_TILE)])

        zeros16 = jnp.zeros((L,), jnp.float32)

        def zero_body(i, _):
            hist_v[pl.ds(i * L, L)] = zeros16
            return 0

        lax.fori_loop(0, N_ACC // L, zero_body, 0)
        plsc.subcore_barrier()

        ones16 = jnp.ones((L,), jnp.float32)
        my_sd = sd_hbm.at[wid]

        def idx_rows(g):
            return my_sd.at[pl.ds(2 * NBUF * g, 2 * NBUF)]

        pltpu.async_copy(idx_rows(0), sd_w.at[0], sem_i)
        pltpu.make_async_copy(idx_rows(0), sd_w.at[0], sem_i).wait()
        pltpu.async_copy(idx_rows(1), sd_w.at[1], sem_i)
        for k in range(NBUF):
            pltpu.async_copy(x_hbm.at[sd_w.at[0, 2 * k]], bufs[k], sems[k])

        def body(j, _):
            s0 = lax.rem(j, 3)
            s1 = lax.rem(j + 1, 3)
            s2 = lax.rem(j + 2, 3)
            pltpu.make_async_copy(idx_rows(j + 1), sd_w.at[s1], sem_i).wait()
            pltpu.async_copy(idx_rows(j + 2), sd_w.at[s2], sem_i)
            for k in range(NBUF):
                pltpu.make_async_copy(x_hbm.at[sd_w.at[s0, 2 * k]], bufs[k],
                                      sems[k]).wait()
                pltpu.sync_copy(bufs[k], acc_sh.at[sd_w.at[s0, 2 * k + 1]],
                                add=True)
                pltpu.async_copy(x_hbm.at[sd_w.at[s1, 2 * k]], bufs[k],
                                 sems[k])
                for m in range(CHUNK // L):
                    d = sd_w[s0, 2 * k + 1, pl.ds(m * L, L)]
                    plsc.addupdate_scatter(hist_v, [d], ones16)
            return 0

        lax.fori_loop(0, GROUPS, body, 0)
        for k in range(NBUF):
            pltpu.make_async_copy(x_hbm.at[sd_w.at[1, 2 * k]], bufs[k],
                                  sems[k]).wait()
        pltpu.make_async_copy(idx_rows(GROUPS + 1),
                              sd_w.at[lax.rem(GROUPS + 1, 3)], sem_i).wait()
        plsc.subcore_barrier()

        pltpu.sync_copy(acc_sh.at[pl.ds(s * ROWS_PER_TILE, ROWS_PER_TILE)],
                        acc_hbm.at[c].at[pl.ds(s * ROWS_PER_TILE,
                                               ROWS_PER_TILE)])
        pltpu.sync_copy(hist_v, deg_hbm.at[c].at[s])

    return sc_kernel(x, sd3, zeros_init)


def _tc_body(x_ref, acc_ref, deg_ref, ws_ref, wn_ref, b_ref, o_ref):
    feats = acc_ref[0] + acc_ref[1]
    deg = jnp.maximum(deg_ref[...], 1.0)
    agg = jnp.dot(feats / deg, wn_ref[...], preferred_element_type=jnp.float32)
    z = jnp.dot(x_ref[...], ws_ref[...], preferred_element_type=jnp.float32)
    o_ref[...] = jnp.maximum(z + agg + b_ref[...], 0.0)


def _tc_finish(x, acc, deg, W_self, W_neigh, b):
    R = 2000
    grid = N_NODES // R
    return pl.pallas_call(
        _tc_body,
        grid=(grid,),
        in_specs=[
            pl.BlockSpec((R, D_FEAT), lambda i: (i, 0)),
            pl.BlockSpec((NC, R, D_FEAT), lambda i: (0, i, 0)),
            pl.BlockSpec((R, 1), lambda i: (i, 0)),
            pl.BlockSpec((D_FEAT, D_FEAT), lambda i: (0, 0)),
            pl.BlockSpec((D_FEAT, D_FEAT), lambda i: (0, 0)),
            pl.BlockSpec((1, D_FEAT), lambda i: (0, 0)),
        ],
        out_specs=pl.BlockSpec((R, D_FEAT), lambda i: (i, 0)),
        out_shape=jax.ShapeDtypeStruct((N_NODES, D_FEAT), jnp.float32),
    )(x, acc, deg, W_self, W_neigh, b.reshape(1, D_FEAT))


def kernel(x, edge_index, batch, W_self, W_neigh, b):
    src = edge_index[0].astype(jnp.int32)
    dst = edge_index[1].astype(jnp.int32)

    pad = E_PAD - N_EDGES
    src_p = jnp.concatenate([src, jnp.zeros((pad,), jnp.int32)])
    dst_p = jnp.concatenate([dst, jnp.full((pad,), JUNK, jnp.int32)])
    chunks = GROUPS * NBUF
    src3 = src_p.reshape(NW, chunks, CHUNK)
    dst3 = dst_p.reshape(NW, chunks, CHUNK)
    extra = SD_GROUPS * NBUF - chunks
    src3 = jnp.concatenate(
        [src3, jnp.zeros((NW, extra, CHUNK), jnp.int32)], axis=1)
    dst3 = jnp.concatenate(
        [dst3, jnp.full((NW, extra, CHUNK), JUNK, jnp.int32)], axis=1)
    sd3 = jnp.stack([src3, dst3], axis=2).reshape(
        NW, 2 * SD_GROUPS * NBUF, CHUNK)

    zeros_init = jnp.zeros((ROWS_PER_TILE, D_FEAT), jnp.float32)

    acc, deg_parts = _sc_accumulate(x, sd3, zeros_init)
    deg = deg_parts.sum(axis=(0, 1))[:N_NODES, None]
    node_emb = _tc_finish(x, acc, deg, W_self, W_neigh, b)
    return node_emb, batch


# speedup vs baseline: 1.3956x; 1.3956x over previous
import functools

import jax
import jax.numpy as jnp
from jax import lax
from jax.experimental import pallas as pl
from jax.experimental.pallas import tpu as pltpu
from jax.experimental.pallas import tpu_sc as plsc

N_NODES = 10000
N_EDGES = 320000
D_FEAT = 128

NC = 2
NS = 16
NW = NC * NS
CHUNK = 128
QA = 21
QB = 59
PAIRS = NS * (QA + QB)
E_PAD = PAIRS * 2 * CHUNK
ROWS_PER_TILE = 632
N_ACC = ROWS_PER_TILE * NS
JUNK = N_NODES
L = 16


def _sc_accumulate(x, sd3, zeros_init):
    mesh = plsc.VectorSubcoreMesh(core_axis_name="c", subcore_axis_name="s")

    @functools.partial(
        pl.kernel,
        out_type=(
            jax.ShapeDtypeStruct((NC, N_ACC, D_FEAT), jnp.float32),
            jax.ShapeDtypeStruct((NC, NS, N_ACC), jnp.float32),
        ),
        mesh=mesh,
        scratch_types=[
            pltpu.VMEM((2, 4, CHUNK), jnp.int32),
            pltpu.VMEM((CHUNK, D_FEAT), jnp.float32),
            pltpu.VMEM((CHUNK, D_FEAT), jnp.float32),
            pltpu.VMEM((N_ACC,), jnp.float32),
            pltpu.VMEM_SHARED((N_ACC, D_FEAT), jnp.float32),
            pltpu.SemaphoreType.DMA,
            pltpu.SemaphoreType.DMA,
            pltpu.SemaphoreType.DMA,
        ],
        compiler_params=pltpu.CompilerParams(needs_layout_passes=False),
    )
    def sc_kernel(x_hbm, sd_hbm, zero_hbm, acc_hbm, deg_hbm,
                  sd_w, rows_a, rows_b, hist_v, acc_sh, sem_a, sem_b, sem_i):
        c = lax.axis_index("c")
        s = lax.axis_index("s")
        base = jnp.where(c == 0, s * QA, NS * QA + s * QB)
        npairs = jnp.where(c == 0, QA, QB)

        pltpu.sync_copy(zero_hbm,
                        acc_sh.at[pl.ds(s * ROWS_PER_TILE, ROWS_PER_TILE)])

        zeros16 = jnp.zeros((L,), jnp.float32)

        def zero_body(i, _):
            hist_v[pl.ds(i * L, L)] = zeros16
            return 0

        lax.fori_loop(0, N_ACC // L, zero_body, 0)
        plsc.subcore_barrier()

        ones16 = jnp.ones((L,), jnp.float32)

        def hist_rows(slot, r):
            for m in range(CHUNK // L):
                d = sd_w[slot, r, pl.ds(m * L, L)]
                plsc.addupdate_scatter(hist_v, [d], ones16)

        pltpu.async_copy(sd_hbm.at[pl.ds(4 * base, 4)], sd_w.at[0], sem_i)

        def body(j, _):
            slot = lax.rem(j, 2)
            nslot = lax.rem(j + 1, 2)
            p = base + j
            pltpu.make_async_copy(sd_hbm.at[pl.ds(4 * p, 4)],
                                  sd_w.at[slot], sem_i).wait()
            pltpu.async_copy(sd_hbm.at[pl.ds(4 * p + 4, 4)],
                             sd_w.at[nslot], sem_i)
            pltpu.async_copy(x_hbm.at[sd_w.at[slot, 0]], rows_a, sem_a)
            pltpu.async_copy(x_hbm.at[sd_w.at[slot, 2]], rows_b, sem_b)
            pltpu.make_async_copy(x_hbm.at[sd_w.at[slot, 0]], rows_a,
                                  sem_a).wait()
            pltpu.sync_copy(rows_a, acc_sh.at[sd_w.at[slot, 1]], add=True)
            hist_rows(slot, 1)
            pltpu.make_async_copy(x_hbm.at[sd_w.at[slot, 2]], rows_b,
                                  sem_b).wait()
            pltpu.sync_copy(rows_b, acc_sh.at[sd_w.at[slot, 3]], add=True)
            hist_rows(slot, 3)
            return 0

        lax.fori_loop(0, npairs, body, 0)
        pltpu.make_async_copy(sd_hbm.at[pl.ds(4 * (base + npairs), 4)],
                              sd_w.at[0], sem_i).wait()
        plsc.subcore_barrier()

        pltpu.sync_copy(acc_sh.at[pl.ds(s * ROWS_PER_TILE, ROWS_PER_TILE)],
                        acc_hbm.at[c].at[pl.ds(s * ROWS_PER_TILE,
                                               ROWS_PER_TILE)])
        pltpu.sync_copy(hist_v, deg_hbm.at[c].at[s])

    return sc_kernel(x, sd3, zeros_init)


def _tc_body(x_ref, acc_ref, deg_ref, ws_ref, wn_ref, b_ref, o_ref):
    feats = acc_ref[0] + acc_ref[1]
    deg = jnp.maximum(deg_ref[...], 1.0)
    agg = jnp.dot(feats / deg, wn_ref[...], preferred_element_type=jnp.float32)
    z = jnp.dot(x_ref[...], ws_ref[...], preferred_element_type=jnp.float32)
    o_ref[...] = jnp.maximum(z + agg + b_ref[...], 0.0)


def _tc_finish(x, acc, deg, W_self, W_neigh, b):
    R = 2000
    grid = N_NODES // R
    return pl.pallas_call(
        _tc_body,
        grid=(grid,),
        in_specs=[
            pl.BlockSpec((R, D_FEAT), lambda i: (i, 0)),
            pl.BlockSpec((NC, R, D_FEAT), lambda i: (0, i, 0)),
            pl.BlockSpec((R, 1), lambda i: (i, 0)),
            pl.BlockSpec((D_FEAT, D_FEAT), lambda i: (0, 0)),
            pl.BlockSpec((D_FEAT, D_FEAT), lambda i: (0, 0)),
            pl.BlockSpec((1, D_FEAT), lambda i: (0, 0)),
        ],
        out_specs=pl.BlockSpec((R, D_FEAT), lambda i: (i, 0)),
        out_shape=jax.ShapeDtypeStruct((N_NODES, D_FEAT), jnp.float32),
    )(x, acc, deg, W_self, W_neigh, b.reshape(1, D_FEAT))


def kernel(x, edge_index, batch, W_self, W_neigh, b):
    src = edge_index[0].astype(jnp.int32)
    dst = edge_index[1].astype(jnp.int32)

    pad = E_PAD - N_EDGES
    src_p = jnp.concatenate([src, jnp.zeros((pad,), jnp.int32)])
    dst_p = jnp.concatenate([dst, jnp.full((pad,), JUNK, jnp.int32)])
    src3 = src_p.reshape(2 * PAIRS, CHUNK)
    dst3 = dst_p.reshape(2 * PAIRS, CHUNK)
    sd3 = jnp.stack([src3, dst3], axis=1).reshape(4 * PAIRS, CHUNK)
    sd3 = jnp.concatenate(
        [sd3, jnp.zeros((4, CHUNK), jnp.int32)], axis=0)

    zeros_init = jnp.zeros((ROWS_PER_TILE, D_FEAT), jnp.float32)

    acc, deg_parts = _sc_accumulate(x, sd3, zeros_init)
    deg = deg_parts.sum(axis=(0, 1))[:N_NODES, None]
    node_emb = _tc_finish(x, acc, deg, W_self, W_neigh, b)
    return node_emb, batch


# speedup vs baseline: 1.6599x; 1.1894x over previous
import functools

import jax
import jax.numpy as jnp
from jax import lax
from jax.experimental import pallas as pl
from jax.experimental.pallas import tpu as pltpu
from jax.experimental.pallas import tpu_sc as plsc

N_NODES = 10000
N_EDGES = 320000
D_FEAT = 128

NC = 2
NS = 16
NW = NC * NS
CHUNK = 128
QA = 59
QB = 21
PAIRS = NS * (QA + QB)
E_PAD = PAIRS * 2 * CHUNK
ROWS_PER_TILE = 632
N_ACC = ROWS_PER_TILE * NS
JUNK = N_NODES
L = 16


def _sc_accumulate(x, sd3, zeros_init):
    mesh = plsc.VectorSubcoreMesh(core_axis_name="c", subcore_axis_name="s")

    @functools.partial(
        pl.kernel,
        out_type=(
            jax.ShapeDtypeStruct((NC, N_ACC, D_FEAT), jnp.float32),
            jax.ShapeDtypeStruct((NC, NS, N_ACC), jnp.float32),
        ),
        mesh=mesh,
        scratch_types=[
            pltpu.VMEM((2, 4, CHUNK), jnp.int32),
            pltpu.VMEM((CHUNK, D_FEAT), jnp.float32),
            pltpu.VMEM((CHUNK, D_FEAT), jnp.float32),
            pltpu.VMEM((N_ACC,), jnp.float32),
            pltpu.VMEM_SHARED((N_ACC, D_FEAT), jnp.float32),
            pltpu.SemaphoreType.DMA,
            pltpu.SemaphoreType.DMA,
            pltpu.SemaphoreType.DMA,
        ],
        compiler_params=pltpu.CompilerParams(needs_layout_passes=False),
    )
    def sc_kernel(x_hbm, sd_hbm, zero_hbm, acc_hbm, deg_hbm,
                  sd_w, rows_a, rows_b, hist_v, acc_sh, sem_a, sem_b, sem_i):
        c = lax.axis_index("c")
        s = lax.axis_index("s")
        base = jnp.where(c == 0, s * QA, NS * QA + s * QB)
        npairs = jnp.where(c == 0, QA, QB)

        pltpu.sync_copy(zero_hbm,
                        acc_sh.at[pl.ds(s * ROWS_PER_TILE, ROWS_PER_TILE)])

        zeros16 = jnp.zeros((L,), jnp.float32)

        def zero_body(i, _):
            hist_v[pl.ds(i * L, L)] = zeros16
            return 0

        lax.fori_loop(0, N_ACC // L, zero_body, 0)
        plsc.subcore_barrier()

        ones16 = jnp.ones((L,), jnp.float32)

        def hist_rows(slot, r):
            for m in range(CHUNK // L):
                d = sd_w[slot, r, pl.ds(m * L, L)]
                plsc.addupdate_scatter(hist_v, [d], ones16)

        pltpu.async_copy(sd_hbm.at[pl.ds(4 * base, 4)], sd_w.at[0], sem_i)

        def body(j, _):
            slot = lax.rem(j, 2)
            nslot = lax.rem(j + 1, 2)
            p = base + j
            pltpu.make_async_copy(sd_hbm.at[pl.ds(4 * p, 4)],
                                  sd_w.at[slot], sem_i).wait()
            pltpu.async_copy(sd_hbm.at[pl.ds(4 * p + 4, 4)],
                             sd_w.at[nslot], sem_i)
            pltpu.async_copy(x_hbm.at[sd_w.at[slot, 0]], rows_a, sem_a)
            pltpu.async_copy(x_hbm.at[sd_w.at[slot, 2]], rows_b, sem_b)
            pltpu.make_async_copy(x_hbm.at[sd_w.at[slot, 0]], rows_a,
                                  sem_a).wait()
            pltpu.sync_copy(rows_a, acc_sh.at[sd_w.at[slot, 1]], add=True)
            hist_rows(slot, 1)
            pltpu.make_async_copy(x_hbm.at[sd_w.at[slot, 2]], rows_b,
                                  sem_b).wait()
            pltpu.sync_copy(rows_b, acc_sh.at[sd_w.at[slot, 3]], add=True)
            hist_rows(slot, 3)
            return 0

        lax.fori_loop(0, npairs, body, 0)
        pltpu.make_async_copy(sd_hbm.at[pl.ds(4 * (base + npairs), 4)],
                              sd_w.at[0], sem_i).wait()
        plsc.subcore_barrier()

        pltpu.sync_copy(acc_sh.at[pl.ds(s * ROWS_PER_TILE, ROWS_PER_TILE)],
                        acc_hbm.at[c].at[pl.ds(s * ROWS_PER_TILE,
                                               ROWS_PER_TILE)])
        pltpu.sync_copy(hist_v, deg_hbm.at[c].at[s])

    return sc_kernel(x, sd3, zeros_init)


def _tc_body(x_ref, acc_ref, deg_ref, ws_ref, wn_ref, b_ref, o_ref):
    feats = acc_ref[0] + acc_ref[1]
    deg = jnp.maximum(deg_ref[...], 1.0)
    agg = jnp.dot(feats / deg, wn_ref[...], preferred_element_type=jnp.float32)
    z = jnp.dot(x_ref[...], ws_ref[...], preferred_element_type=jnp.float32)
    o_ref[...] = jnp.maximum(z + agg + b_ref[...], 0.0)


def _tc_finish(x, acc, deg, W_self, W_neigh, b):
    R = 2000
    grid = N_NODES // R
    return pl.pallas_call(
        _tc_body,
        grid=(grid,),
        in_specs=[
            pl.BlockSpec((R, D_FEAT), lambda i: (i, 0)),
            pl.BlockSpec((NC, R, D_FEAT), lambda i: (0, i, 0)),
            pl.BlockSpec((R, 1), lambda i: (i, 0)),
            pl.BlockSpec((D_FEAT, D_FEAT), lambda i: (0, 0)),
            pl.BlockSpec((D_FEAT, D_FEAT), lambda i: (0, 0)),
            pl.BlockSpec((1, D_FEAT), lambda i: (0, 0)),
        ],
        out_specs=pl.BlockSpec((R, D_FEAT), lambda i: (i, 0)),
        out_shape=jax.ShapeDtypeStruct((N_NODES, D_FEAT), jnp.float32),
    )(x, acc, deg, W_self, W_neigh, b.reshape(1, D_FEAT))


def kernel(x, edge_index, batch, W_self, W_neigh, b):
    src = edge_index[0].astype(jnp.int32)
    dst = edge_index[1].astype(jnp.int32)

    pad = E_PAD - N_EDGES
    src_p = jnp.concatenate([src, jnp.zeros((pad,), jnp.int32)])
    dst_p = jnp.concatenate([dst, jnp.full((pad,), JUNK, jnp.int32)])
    src3 = src_p.reshape(2 * PAIRS, CHUNK)
    dst3 = dst_p.reshape(2 * PAIRS, CHUNK)
    sd3 = jnp.stack([src3, dst3], axis=1).reshape(4 * PAIRS, CHUNK)
    sd3 = jnp.concatenate(
        [sd3, jnp.zeros((4, CHUNK), jnp.int32)], axis=0)

    zeros_init = jnp.zeros((ROWS_PER_TILE, D_FEAT), jnp.float32)

    acc, deg_parts = _sc_accumulate(x, sd3, zeros_init)
    deg = deg_parts.sum(axis=(0, 1))[:N_NODES, None]
    node_emb = _tc_finish(x, acc, deg, W_self, W_neigh, b)
    return node_emb, batch


# speedup vs baseline: 2.9134x; 1.7551x over previous
import functools

import jax
import jax.numpy as jnp
from jax import lax
from jax.experimental import pallas as pl
from jax.experimental.pallas import tpu as pltpu
from jax.experimental.pallas import tpu_sc as plsc

N_NODES = 10000
N_EDGES = 320000
D_FEAT = 128

NC = 2
NS = 16
CHUNK = 128
PAIRS = N_EDGES // (2 * CHUNK)
CORE0_PAIRS = 919
CORE1_PAIRS = PAIRS - CORE0_PAIRS
Q0, R0 = divmod(CORE0_PAIRS, NS)
Q1, R1 = divmod(CORE1_PAIRS, NS)
ROWS_PER_TILE = 632
N_ACC = ROWS_PER_TILE * NS
L = 16


def _sc_accumulate(x, e2, zeros_init):
    mesh = plsc.VectorSubcoreMesh(core_axis_name="c", subcore_axis_name="s")

    @functools.partial(
        pl.kernel,
        out_type=(
            jax.ShapeDtypeStruct((NC, N_ACC, D_FEAT), jnp.float32),
            jax.ShapeDtypeStruct((NC, NS, N_ACC), jnp.float32),
        ),
        mesh=mesh,
        scratch_types=[
            pltpu.VMEM((2, 2, CHUNK), jnp.int32),
            pltpu.VMEM((2, 2, CHUNK), jnp.int32),
            pltpu.VMEM((CHUNK, D_FEAT), jnp.float32),
            pltpu.VMEM((CHUNK, D_FEAT), jnp.float32),
            pltpu.VMEM((N_ACC,), jnp.float32),
            pltpu.VMEM_SHARED((N_ACC, D_FEAT), jnp.float32),
            pltpu.SemaphoreType.DMA,
            pltpu.SemaphoreType.DMA,
            pltpu.SemaphoreType.DMA,
            pltpu.SemaphoreType.DMA,
        ],
        compiler_params=pltpu.CompilerParams(needs_layout_passes=False),
    )
    def sc_kernel(x_hbm, sd_hbm, zero_hbm, acc_hbm, deg_hbm,
                  src_w, dst_w, rows_a, rows_b, hist_v, acc_sh,
                  sem_a, sem_b, sem_is, sem_id):
        c = lax.axis_index("c")
        s = lax.axis_index("s")
        base = jnp.where(
            c == 0,
            s * Q0 + jnp.minimum(s, R0),
            CORE0_PAIRS + s * Q1 + jnp.minimum(s, R1),
        )
        npairs = jnp.where(c == 0,
                           Q0 + (s < R0).astype(jnp.int32),
                           Q1 + (s < R1).astype(jnp.int32))

        pltpu.sync_copy(zero_hbm,
                        acc_sh.at[pl.ds(s * ROWS_PER_TILE, ROWS_PER_TILE)])

        zeros16 = jnp.zeros((L,), jnp.float32)

        def zero_body(i, _):
            hist_v[pl.ds(i * L, L)] = zeros16
            return 0

        lax.fori_loop(0, N_ACC // L, zero_body, 0)
        plsc.subcore_barrier()

        ones16 = jnp.ones((L,), jnp.float32)

        def hist_rows(slot, r):
            for m in range(CHUNK // L):
                d = dst_w[slot, r, pl.ds(m * L, L)]
                plsc.addupdate_scatter(hist_v, [d], ones16)

        def fetch_idx(p, slot, wait):
            rows = pl.ds(2 * p, 2)
            if wait:
                pltpu.make_async_copy(sd_hbm.at[0].at[rows], src_w.at[slot],
                                      sem_is).wait()
                pltpu.make_async_copy(sd_hbm.at[1].at[rows], dst_w.at[slot],
                                      sem_id).wait()
            else:
                pltpu.async_copy(sd_hbm.at[0].at[rows], src_w.at[slot],
                                 sem_is)
                pltpu.async_copy(sd_hbm.at[1].at[rows], dst_w.at[slot],
                                 sem_id)

        fetch_idx(base, 0, False)

        def body(j, _):
            slot = lax.rem(j, 2)
            nslot = lax.rem(j + 1, 2)
            fetch_idx(base + j, slot, True)
            fetch_idx(jnp.minimum(base + j + 1, PAIRS - 1), nslot, False)
            pltpu.async_copy(x_hbm.at[src_w.at[slot, 0]], rows_a, sem_a)
            pltpu.async_copy(x_hbm.at[src_w.at[slot, 1]], rows_b, sem_b)
            pltpu.make_async_copy(x_hbm.at[src_w.at[slot, 0]], rows_a,
                                  sem_a).wait()
            pltpu.sync_copy(rows_a, acc_sh.at[dst_w.at[slot, 0]], add=True)
            hist_rows(slot, 0)
            pltpu.make_async_copy(x_hbm.at[src_w.at[slot, 1]], rows_b,
                                  sem_b).wait()
            pltpu.sync_copy(rows_b, acc_sh.at[dst_w.at[slot, 1]], add=True)
            hist_rows(slot, 1)
            return 0

        lax.fori_loop(0, npairs, body, 0)
        fetch_idx(PAIRS - 1, lax.rem(npairs, 2), True)
        plsc.subcore_barrier()

        pltpu.sync_copy(acc_sh.at[pl.ds(s * ROWS_PER_TILE, ROWS_PER_TILE)],
                        acc_hbm.at[c].at[pl.ds(s * ROWS_PER_TILE,
                                               ROWS_PER_TILE)])
        pltpu.sync_copy(hist_v, deg_hbm.at[c].at[s])

    return sc_kernel(x, e2, zeros_init)


def _tc_body(x_ref, acc_ref, deg_ref, ws_ref, wn_ref, b_ref, o_ref):
    feats = acc_ref[0] + acc_ref[1]
    deg = jnp.maximum(deg_ref[...], 1.0)
    agg = jnp.dot(feats / deg, wn_ref[...], preferred_element_type=jnp.float32)
    z = jnp.dot(x_ref[...], ws_ref[...], preferred_element_type=jnp.float32)
    o_ref[...] = jnp.maximum(z + agg + b_ref[...], 0.0)


def _tc_finish(x, acc, deg, W_self, W_neigh, b):
    R = 2000
    grid = N_NODES // R
    return pl.pallas_call(
        _tc_body,
        grid=(grid,),
        in_specs=[
            pl.BlockSpec((R, D_FEAT), lambda i: (i, 0)),
            pl.BlockSpec((NC, R, D_FEAT), lambda i: (0, i, 0)),
            pl.BlockSpec((R, 1), lambda i: (i, 0)),
            pl.BlockSpec((D_FEAT, D_FEAT), lambda i: (0, 0)),
            pl.BlockSpec((D_FEAT, D_FEAT), lambda i: (0, 0)),
            pl.BlockSpec((1, D_FEAT), lambda i: (0, 0)),
        ],
        out_specs=pl.BlockSpec((R, D_FEAT), lambda i: (i, 0)),
        out_shape=jax.ShapeDtypeStruct((N_NODES, D_FEAT), jnp.float32),
    )(x, acc, deg, W_self, W_neigh, b.reshape(1, D_FEAT))


def kernel(x, edge_index, batch, W_self, W_neigh, b):
    e2 = edge_index.astype(jnp.int32).reshape(2, 2 * PAIRS, CHUNK)
    zeros_init = jnp.zeros((ROWS_PER_TILE, D_FEAT), jnp.float32)

    acc, deg_parts = _sc_accumulate(x, e2, zeros_init)
    deg = deg_parts.sum(axis=(0, 1))[:N_NODES, None]
    node_emb = _tc_finish(x, acc, deg, W_self, W_neigh, b)
    return node_emb, batch


# speedup vs baseline: 2.9593x; 1.0158x over previous
import functools

import jax
import jax.numpy as jnp
from jax import lax
from jax.experimental import pallas as pl
from jax.experimental.pallas import tpu as pltpu
from jax.experimental.pallas import tpu_sc as plsc

N_NODES = 10000
N_EDGES = 320000
D_FEAT = 128

NC = 2
NS = 16
CHUNK = 256
PAIRS = N_EDGES // CHUNK
CORE0_PAIRS = 919
CORE1_PAIRS = PAIRS - CORE0_PAIRS
Q0, R0 = divmod(CORE0_PAIRS, NS)
Q1, R1 = divmod(CORE1_PAIRS, NS)
ROWS_PER_TILE = 632
N_ACC = ROWS_PER_TILE * NS
L = 16


def _sc_accumulate(x, e2, zeros_init):
    mesh = plsc.VectorSubcoreMesh(core_axis_name="c", subcore_axis_name="s")

    @functools.partial(
        pl.kernel,
        out_type=(
            jax.ShapeDtypeStruct((NC, N_ACC, D_FEAT), jnp.float32),
            jax.ShapeDtypeStruct((NC, NS, N_ACC), jnp.float32),
        ),
        mesh=mesh,
        scratch_types=[
            pltpu.VMEM((2, 1, CHUNK), jnp.int32),
            pltpu.VMEM((2, 1, CHUNK), jnp.int32),
            pltpu.VMEM((CHUNK, D_FEAT), jnp.float32),
            pltpu.VMEM((N_ACC,), jnp.float32),
            pltpu.VMEM_SHARED((N_ACC, D_FEAT), jnp.float32),
            pltpu.SemaphoreType.DMA,
            pltpu.SemaphoreType.DMA,
            pltpu.SemaphoreType.DMA,
            pltpu.SemaphoreType.DMA,
        ],
        compiler_params=pltpu.CompilerParams(needs_layout_passes=False),
    )
    def sc_kernel(x_hbm, sd_hbm, zero_hbm, acc_hbm, deg_hbm,
                  src_w, dst_w, rows_a, hist_v, acc_sh,
                  sem_a, sem_b, sem_is, sem_id):
        c = lax.axis_index("c")
        s = lax.axis_index("s")
        base = jnp.where(
            c == 0,
            s * Q0 + jnp.minimum(s, R0),
            CORE0_PAIRS + s * Q1 + jnp.minimum(s, R1),
        )
        npairs = jnp.where(c == 0,
                           Q0 + (s < R0).astype(jnp.int32),
                           Q1 + (s < R1).astype(jnp.int32))

        pltpu.sync_copy(zero_hbm,
                        acc_sh.at[pl.ds(s * ROWS_PER_TILE, ROWS_PER_TILE)])

        zeros16 = jnp.zeros((L,), jnp.float32)

        def zero_body(i, _):
            hist_v[pl.ds(i * L, L)] = zeros16
            return 0

        lax.fori_loop(0, N_ACC // L, zero_body, 0)
        plsc.subcore_barrier()

        ones16 = jnp.ones((L,), jnp.float32)

        def hist_rows(slot, r):
            for m in range(CHUNK // L):
                d = dst_w[slot, r, pl.ds(m * L, L)]
                plsc.addupdate_scatter(hist_v, [d], ones16)

        def fetch_idx(p, slot, wait):
            rows = pl.ds(p, 1)
            if wait:
                pltpu.make_async_copy(sd_hbm.at[0].at[rows], src_w.at[slot],
                                      sem_is).wait()
                pltpu.make_async_copy(sd_hbm.at[1].at[rows], dst_w.at[slot],
                                      sem_id).wait()
            else:
                pltpu.async_copy(sd_hbm.at[0].at[rows], src_w.at[slot],
                                 sem_is)
                pltpu.async_copy(sd_hbm.at[1].at[rows], dst_w.at[slot],
                                 sem_id)

        fetch_idx(base, 0, False)

        def body(j, _):
            slot = lax.rem(j, 2)
            nslot = lax.rem(j + 1, 2)
            fetch_idx(base + j, slot, True)
            fetch_idx(jnp.minimum(base + j + 1, PAIRS - 1), nslot, False)
            pltpu.async_copy(x_hbm.at[src_w.at[slot, 0]], rows_a, sem_a)
            hist_rows(slot, 0)
            pltpu.make_async_copy(x_hbm.at[src_w.at[slot, 0]], rows_a,
                                  sem_a).wait()
            pltpu.sync_copy(rows_a, acc_sh.at[dst_w.at[slot, 0]], add=True)
            return 0

        lax.fori_loop(0, npairs, body, 0)
        fetch_idx(PAIRS - 1, lax.rem(npairs, 2), True)
        plsc.subcore_barrier()

        pltpu.sync_copy(acc_sh.at[pl.ds(s * ROWS_PER_TILE, ROWS_PER_TILE)],
                        acc_hbm.at[c].at[pl.ds(s * ROWS_PER_TILE,
                                               ROWS_PER_TILE)])
        pltpu.sync_copy(hist_v, deg_hbm.at[c].at[s])

    return sc_kernel(x, e2, zeros_init)


def _tc_body(x_ref, acc_ref, deg_ref, ws_ref, wn_ref, b_ref, o_ref):
    feats = acc_ref[0] + acc_ref[1]
    deg = jnp.maximum(deg_ref[...], 1.0)
    agg = jnp.dot(feats / deg, wn_ref[...], preferred_element_type=jnp.float32)
    z = jnp.dot(x_ref[...], ws_ref[...], preferred_element_type=jnp.float32)
    o_ref[...] = jnp.maximum(z + agg + b_ref[...], 0.0)


def _tc_finish(x, acc, deg, W_self, W_neigh, b):
    R = 2000
    grid = N_NODES // R
    return pl.pallas_call(
        _tc_body,
        grid=(grid,),
        in_specs=[
            pl.BlockSpec((R, D_FEAT), lambda i: (i, 0)),
            pl.BlockSpec((NC, R, D_FEAT), lambda i: (0, i, 0)),
            pl.BlockSpec((R, 1), lambda i: (i, 0)),
            pl.BlockSpec((D_FEAT, D_FEAT), lambda i: (0, 0)),
            pl.BlockSpec((D_FEAT, D_FEAT), lambda i: (0, 0)),
            pl.BlockSpec((1, D_FEAT), lambda i: (0, 0)),
        ],
        out_specs=pl.BlockSpec((R, D_FEAT), lambda i: (i, 0)),
        out_shape=jax.ShapeDtypeStruct((N_NODES, D_FEAT), jnp.float32),
    )(x, acc, deg, W_self, W_neigh, b.reshape(1, D_FEAT))


def kernel(x, edge_index, batch, W_self, W_neigh, b):
    e2 = edge_index.astype(jnp.int32).reshape(2, PAIRS, CHUNK)
    zeros_init = jnp.zeros((ROWS_PER_TILE, D_FEAT), jnp.float32)

    acc, deg_parts = _sc_accumulate(x, e2, zeros_init)
    deg = deg_parts.sum(axis=(0, 1))[:N_NODES, None]
    node_emb = _tc_finish(x, acc, deg, W_self, W_neigh, b)
    return node_emb, batch


# speedup vs baseline: 3.6699x; 1.2401x over previous
import functools

import jax
import jax.numpy as jnp
from jax import lax
from jax.experimental import pallas as pl
from jax.experimental.pallas import tpu as pltpu
from jax.experimental.pallas import tpu_sc as plsc

N_NODES = 10000
N_EDGES = 320000
D_FEAT = 128

NC = 2
NS = 16
CHUNK = 256
PAIRS = N_EDGES // CHUNK
CORE0_PAIRS = 680
CORE1_PAIRS = PAIRS - CORE0_PAIRS
Q0, R0 = divmod(CORE0_PAIRS, NS)
Q1, R1 = divmod(CORE1_PAIRS, NS)
ROWS_PER_TILE = 632
N_ACC = ROWS_PER_TILE * NS
L = 16


def _sc_accumulate(x, e2, zeros_init):
    mesh = plsc.VectorSubcoreMesh(core_axis_name="c", subcore_axis_name="s")

    @functools.partial(
        pl.kernel,
        out_type=(
            jax.ShapeDtypeStruct((NC, N_ACC, D_FEAT), jnp.float32),
            jax.ShapeDtypeStruct((NC, NS, N_ACC), jnp.float32),
        ),
        mesh=mesh,
        scratch_types=[
            pltpu.VMEM((2, 1, CHUNK), jnp.int32),
            pltpu.VMEM((2, 1, CHUNK), jnp.int32),
            pltpu.VMEM((CHUNK, D_FEAT), jnp.float32),
            pltpu.VMEM((N_ACC,), jnp.float32),
            pltpu.VMEM_SHARED((N_ACC, D_FEAT), jnp.float32),
            pltpu.SemaphoreType.DMA,
            pltpu.SemaphoreType.DMA,
            pltpu.SemaphoreType.DMA,
            pltpu.SemaphoreType.DMA,
        ],
        compiler_params=pltpu.CompilerParams(needs_layout_passes=False),
    )
    def sc_kernel(x_hbm, sd_hbm, zero_hbm, acc_hbm, deg_hbm,
                  src_w, dst_w, rows_a, hist_v, acc_sh,
                  sem_a, sem_b, sem_is, sem_id):
        c = lax.axis_index("c")
        s = lax.axis_index("s")
        base = jnp.where(
            c == 0,
            s * Q0 + jnp.minimum(s, R0),
            CORE0_PAIRS + s * Q1 + jnp.minimum(s, R1),
        )
        npairs = jnp.where(c == 0,
                           Q0 + (s < R0).astype(jnp.int32),
                           Q1 + (s < R1).astype(jnp.int32))

        pltpu.sync_copy(zero_hbm,
                        acc_sh.at[pl.ds(s * ROWS_PER_TILE, ROWS_PER_TILE)])

        zeros16 = jnp.zeros((L,), jnp.float32)

        def zero_body(i, _):
            hist_v[pl.ds(i * L, L)] = zeros16
            return 0

        lax.fori_loop(0, N_ACC // L, zero_body, 0)
        plsc.subcore_barrier()

        ones16 = jnp.ones((L,), jnp.float32)

        def hist_rows(slot, r):
            for m in range(CHUNK // L):
                d = dst_w[slot, r, pl.ds(m * L, L)]
                plsc.addupdate_scatter(hist_v, [d], ones16)

        def fetch_idx(p, slot, wait):
            rows = pl.ds(p, 1)
            if wait:
                pltpu.make_async_copy(sd_hbm.at[0].at[rows], src_w.at[slot],
                                      sem_is).wait()
                pltpu.make_async_copy(sd_hbm.at[1].at[rows], dst_w.at[slot],
                                      sem_id).wait()
            else:
                pltpu.async_copy(sd_hbm.at[0].at[rows], src_w.at[slot],
                                 sem_is)
                pltpu.async_copy(sd_hbm.at[1].at[rows], dst_w.at[slot],
                                 sem_id)

        fetch_idx(base, 0, False)

        def body(j, _):
            slot = lax.rem(j, 2)
            nslot = lax.rem(j + 1, 2)
            fetch_idx(base + j, slot, True)
            fetch_idx(jnp.minimum(base + j + 1, PAIRS - 1), nslot, False)
            pltpu.async_copy(x_hbm.at[src_w.at[slot, 0]], rows_a, sem_a)
            hist_rows(slot, 0)
            pltpu.make_async_copy(x_hbm.at[src_w.at[slot, 0]], rows_a,
                                  sem_a).wait()
            pltpu.sync_copy(rows_a, acc_sh.at[dst_w.at[slot, 0]], add=True)
            return 0

        lax.fori_loop(0, npairs, body, 0)
        fetch_idx(PAIRS - 1, lax.rem(npairs, 2), True)
        plsc.subcore_barrier()

        pltpu.sync_copy(acc_sh.at[pl.ds(s * ROWS_PER_TILE, ROWS_PER_TILE)],
                        acc_hbm.at[c].at[pl.ds(s * ROWS_PER_TILE,
                                               ROWS_PER_TILE)])
        pltpu.sync_copy(hist_v, deg_hbm.at[c].at[s])

    return sc_kernel(x, e2, zeros_init)


def _tc_body(x_ref, acc_ref, deg_ref, ws_ref, wn_ref, b_ref, o_ref):
    feats = acc_ref[0] + acc_ref[1]
    deg = jnp.maximum(deg_ref[...], 1.0)
    agg = jnp.dot(feats / deg, wn_ref[...], preferred_element_type=jnp.float32)
    z = jnp.dot(x_ref[...], ws_ref[...], preferred_element_type=jnp.float32)
    o_ref[...] = jnp.maximum(z + agg + b_ref[...], 0.0)


def _tc_finish(x, acc, deg, W_self, W_neigh, b):
    R = 2000
    grid = N_NODES // R
    return pl.pallas_call(
        _tc_body,
        grid=(grid,),
        in_specs=[
            pl.BlockSpec((R, D_FEAT), lambda i: (i, 0)),
            pl.BlockSpec((NC, R, D_FEAT), lambda i: (0, i, 0)),
            pl.BlockSpec((R, 1), lambda i: (i, 0)),
            pl.BlockSpec((D_FEAT, D_FEAT), lambda i: (0, 0)),
            pl.BlockSpec((D_FEAT, D_FEAT), lambda i: (0, 0)),
            pl.BlockSpec((1, D_FEAT), lambda i: (0, 0)),
        ],
        out_specs=pl.BlockSpec((R, D_FEAT), lambda i: (i, 0)),
        out_shape=jax.ShapeDtypeStruct((N_NODES, D_FEAT), jnp.float32),
    )(x, acc, deg, W_self, W_neigh, b.reshape(1, D_FEAT))


def kernel(x, edge_index, batch, W_self, W_neigh, b):
    e2 = edge_index.astype(jnp.int32).reshape(2, PAIRS, CHUNK)
    zeros_init = jnp.zeros((ROWS_PER_TILE, D_FEAT), jnp.float32)

    acc, deg_parts = _sc_accumulate(x, e2, zeros_init)
    deg = deg_parts.sum(axis=(0, 1))[:N_NODES, None]
    node_emb = _tc_finish(x, acc, deg, W_self, W_neigh, b)
    return node_emb, batch
